# baseline (speedup 1.0000x reference)
import functools

import jax
import jax.numpy as jnp
from jax import lax
from jax.experimental import pallas as pl
from jax.experimental.pallas import tpu as pltpu
from jax.experimental.pallas import tpu_sc as plsc

B = 4096
DENSE = 8
N_SPARSE = 10
SPARSE_VOCAB = 100000
D = 32
MAXLEN = 50
M_ITEMS = 100

NC = 2
NS = 16
NW = NC * NS

SEQ_ROWS = B * MAXLEN
SEQ_PER_W = SEQ_ROWS // NW
SEQ_CHUNK = 1600
SPARSE_ROWS = B * N_SPARSE
SPARSE_PER_W = SPARSE_ROWS // NW
ITEM_PAD = 128


def _sc_gather_body(seq_idx, sparse_idx, item_idx, table_seq, table_sparse,
                    seq_out, sparse_out, item_out,
                    idx_v, rows_v, sidx_v, srows_v, iidx_v, irows_v, sem):
    wid = lax.axis_index("s") * NC + lax.axis_index("c")
    base = wid * SEQ_PER_W
    for ci in range(SEQ_PER_W // SEQ_CHUNK):
        off = base + ci * SEQ_CHUNK
        pltpu.sync_copy(seq_idx.at[pl.ds(off, SEQ_CHUNK)], idx_v)
        pltpu.async_copy(table_seq.at[idx_v], rows_v, sem).wait()
        pltpu.sync_copy(rows_v, seq_out.at[pl.ds(off, SEQ_CHUNK)])
    sbase = wid * SPARSE_PER_W
    pltpu.sync_copy(sparse_idx.at[pl.ds(sbase, SPARSE_PER_W)], sidx_v)
    pltpu.async_copy(table_sparse.at[sidx_v], srows_v, sem).wait()
    pltpu.sync_copy(srows_v, sparse_out.at[pl.ds(sbase, SPARSE_PER_W)])
    @pl.when(wid == 0)
    def _():
        pltpu.sync_copy(item_idx.at[pl.ds(0, ITEM_PAD)], iidx_v)
        pltpu.async_copy(table_seq.at[iidx_v], irows_v, sem).wait()
        pltpu.sync_copy(irows_v, item_out.at[pl.ds(0, ITEM_PAD)])


@functools.lru_cache(maxsize=1)
def _get_sc_gather():
  return pl.kernel(
    _sc_gather_body,
    mesh=plsc.VectorSubcoreMesh(core_axis_name="c", subcore_axis_name="s"),
    out_type=[
        jax.ShapeDtypeStruct((SEQ_ROWS, D), jnp.float32),
        jax.ShapeDtypeStruct((SPARSE_ROWS, D), jnp.float32),
        jax.ShapeDtypeStruct((ITEM_PAD, D), jnp.float32),
    ],
    scratch_types=[
        pltpu.VMEM((SEQ_CHUNK,), jnp.int32),
        pltpu.VMEM((SEQ_CHUNK, D), jnp.float32),
        pltpu.VMEM((SPARSE_PER_W,), jnp.int32),
        pltpu.VMEM((SPARSE_PER_W, D), jnp.float32),
        pltpu.VMEM((ITEM_PAD,), jnp.int32),
        pltpu.VMEM((ITEM_PAD, D), jnp.float32),
        pltpu.SemaphoreType.DMA,
    ],
    compiler_params=pltpu.CompilerParams(use_tc_tiling_on_sc=False),
  )


BB = 128
G = B // BB


def _tc_dense_body(seq_ref, dense_ref, sparse_ref, item_ref,
                   w0_ref, w1_ref, w2_ref, w3_ref, b_ref,
                   f1m_ref, f1d_ref, f1s_ref, f1b_ref,
                   f2m_ref, f2d_ref, f2s_ref, f2b_ref,
                   out_ref):
    S = seq_ref[...]
    m_s = jnp.mean(S, axis=0)
    m_t = S[MAXLEN - 1]
    c = m_s @ w2_ref[...] + m_t @ w3_ref[...] + b_ref[...]
    S2 = S.reshape(MAXLEN * BB, D)
    E = (S2 @ w1_ref[...]).reshape(MAXLEN, BB, D) + c[None, :, :]
    att = jax.nn.sigmoid(E)
    alpha = jnp.sum(att * w0_ref[...][None, :, :], axis=-1, keepdims=True)
    m_a = jnp.sum(alpha * S, axis=0)

    xd = dense_ref[...]
    xs = sparse_ref[...]
    h_s = jnp.tanh(m_a @ f1m_ref[...] + xd @ f1d_ref[...] + xs @ f1s_ref[...]
                   + f1b_ref[...])
    h_t = jnp.tanh(m_t @ f2m_ref[...] + xd @ f2d_ref[...] + xs @ f2s_ref[...]
                   + f2b_ref[...])
    p = h_s * h_t
    z = lax.dot_general(p, item_ref[...], (((1,), (1,)), ((), ())))
    z = z[:, :M_ITEMS]
    z = z - jnp.max(z, axis=-1, keepdims=True)
    ez = jnp.exp(z)
    out_ref[...] = ez / jnp.sum(ez, axis=-1, keepdims=True)


def _full_spec(shape):
    return pl.BlockSpec(shape, lambda i: tuple(0 for _ in shape))


_tc_dense = pl.pallas_call(
    _tc_dense_body,
    grid=(G,),
    in_specs=[
        pl.BlockSpec((MAXLEN, BB, D), lambda i: (0, i, 0)),
        pl.BlockSpec((BB, DENSE), lambda i: (i, 0)),
        pl.BlockSpec((BB, N_SPARSE * D), lambda i: (i, 0)),
        _full_spec((ITEM_PAD, D)),
        _full_spec((1, D)),
        _full_spec((D, D)),
        _full_spec((D, D)),
        _full_spec((D, D)),
        _full_spec((1, D)),
        _full_spec((D, D)),
        _full_spec((DENSE, D)),
        _full_spec((N_SPARSE * D, D)),
        _full_spec((1, D)),
        _full_spec((D, D)),
        _full_spec((DENSE, D)),
        _full_spec((N_SPARSE * D, D)),
        _full_spec((1, D)),
    ],
    out_specs=pl.BlockSpec((BB, M_ITEMS), lambda i: (i, 0)),
    out_shape=jax.ShapeDtypeStruct((B, M_ITEMS), jnp.float32),
)


@jax.jit
def kernel(dense_inputs, sparse_inputs, seq_inputs, item_pooling, table_sparse,
           table_seq, W0, W1, W2, W3, b, ffn1_W, ffn1_b, ffn2_W, ffn2_b):
    seq_idx = seq_inputs[:, 0, :].T.reshape(-1)
    feat_off = (jnp.arange(N_SPARSE, dtype=jnp.int32) * SPARSE_VOCAB)[None, :]
    sparse_idx = (sparse_inputs + feat_off).reshape(-1)
    item_idx = jnp.concatenate(
        [item_pooling[:, 0], jnp.zeros((ITEM_PAD - M_ITEMS,), jnp.int32)])
    table_sparse_flat = table_sparse.reshape(N_SPARSE * SPARSE_VOCAB, D)

    seq_flat, sparse_flat, item_embed = _get_sc_gather()(
        seq_idx, sparse_idx, item_idx, table_seq, table_sparse_flat)

    seq_embed = seq_flat.reshape(MAXLEN, B, D)
    sparse_embed = sparse_flat.reshape(B, N_SPARSE * D)

    return _tc_dense(
        seq_embed, dense_inputs, sparse_embed, item_embed,
        W0.T, W1, W2, W3, b[None, :],
        ffn1_W[:D], ffn1_W[D:D + DENSE], ffn1_W[D + DENSE:], ffn1_b[None, :],
        ffn2_W[:D], ffn2_W[D:D + DENSE], ffn2_W[D + DENSE:], ffn2_b[None, :])


# speedup vs baseline: 1.0529x; 1.0529x over previous
import functools

import jax
import jax.numpy as jnp
from jax import lax
from jax.experimental import pallas as pl
from jax.experimental.pallas import tpu as pltpu
from jax.experimental.pallas import tpu_sc as plsc

B = 4096
DENSE = 8
N_SPARSE = 10
SPARSE_VOCAB = 100000
D = 32
MAXLEN = 50
M_ITEMS = 100

NC = 2
NS = 16
NW = NC * NS

SEQ_ROWS = B * MAXLEN
SEQ_PER_W = SEQ_ROWS // NW
SEQ_CHUNK = 1600
SPARSE_ROWS = B * N_SPARSE
SPARSE_PER_W = SPARSE_ROWS // NW
ITEM_PAD = 128

GQ = 4
BL = B // GQ


def _sc_gather_body(seq_idx, sparse_idx, item_idx, table_seq, table_sparse,
                    seq_out, sparse_out, item_out,
                    idx_v, rows_v, sidx_v, srows_v, iidx_v, irows_v, sem):
    wid = lax.axis_index("s") * NC + lax.axis_index("c")
    base = wid * SEQ_PER_W
    for ci in range(SEQ_PER_W // SEQ_CHUNK):
        off = base + ci * SEQ_CHUNK
        pltpu.sync_copy(seq_idx.at[pl.ds(off, SEQ_CHUNK)], idx_v)
        pltpu.async_copy(table_seq.at[idx_v], rows_v, sem).wait()
        pltpu.sync_copy(rows_v, seq_out.at[pl.ds(off, SEQ_CHUNK)])
    sbase = wid * SPARSE_PER_W
    pltpu.sync_copy(sparse_idx.at[pl.ds(sbase, SPARSE_PER_W)], sidx_v)
    pltpu.async_copy(table_sparse.at[sidx_v], srows_v, sem).wait()
    pltpu.sync_copy(srows_v, sparse_out.at[pl.ds(sbase, SPARSE_PER_W)])
    @pl.when(wid == 0)
    def _():
        pltpu.sync_copy(item_idx.at[pl.ds(0, ITEM_PAD)], iidx_v)
        pltpu.async_copy(table_seq.at[iidx_v], irows_v, sem).wait()
        pltpu.sync_copy(irows_v, item_out.at[pl.ds(0, ITEM_PAD)])


@functools.lru_cache(maxsize=1)
def _get_sc_gather():
  return pl.kernel(
    _sc_gather_body,
    mesh=plsc.VectorSubcoreMesh(core_axis_name="c", subcore_axis_name="s"),
    out_type=[
        jax.ShapeDtypeStruct((SEQ_ROWS, D), jnp.float32),
        jax.ShapeDtypeStruct((SPARSE_ROWS, D), jnp.float32),
        jax.ShapeDtypeStruct((ITEM_PAD, D), jnp.float32),
    ],
    scratch_types=[
        pltpu.VMEM((SEQ_CHUNK,), jnp.int32),
        pltpu.VMEM((SEQ_CHUNK, D), jnp.float32),
        pltpu.VMEM((SPARSE_PER_W,), jnp.int32),
        pltpu.VMEM((SPARSE_PER_W, D), jnp.float32),
        pltpu.VMEM((ITEM_PAD,), jnp.int32),
        pltpu.VMEM((ITEM_PAD, D), jnp.float32),
        pltpu.SemaphoreType.DMA,
    ],
    compiler_params=pltpu.CompilerParams(use_tc_tiling_on_sc=False),
  )


BB = 256
GB = BB // GQ
G = B // BB


def _tc_dense_body(seq_ref, dense_ref, sparse_ref, item_ref, sel_ref,
                   w0_ref, w1_ref, w2_ref, w3_ref, b_ref, gones_ref,
                   f1m_ref, f1d_ref, f1s_ref, f1b_ref,
                   f2m_ref, f2d_ref, f2s_ref, f2b_ref,
                   out_ref):
    S = seq_ref[...]
    m_s = jnp.mean(S, axis=0)
    m_t = S[MAXLEN - 1]
    c = m_s @ w2_ref[...] + m_t @ w3_ref[...] + b_ref[...]
    S2 = S.reshape(MAXLEN * GB, 128)
    E = (S2 @ w1_ref[...]).reshape(MAXLEN, GB, 128) + c[None, :, :]
    att = jax.nn.sigmoid(E)
    aw = att * w0_ref[...][None, :, :]
    alpha = (aw.reshape(MAXLEN * GB, 128) @ gones_ref[...]).reshape(
        MAXLEN, GB, 128)
    m_a = jnp.sum(alpha * S, axis=0)

    xd = dense_ref[...]
    acc1 = m_a @ f1m_ref[...] + xd @ f1d_ref[...] + f1b_ref[...]
    acc2 = m_t @ f2m_ref[...] + xd @ f2d_ref[...] + f2b_ref[...]
    for i in range(N_SPARSE):
        xi = sparse_ref[i]
        acc1 = acc1 + xi @ f1s_ref[i]
        acc2 = acc2 + xi @ f2s_ref[i]
    p4 = jnp.tanh(acc1) * jnp.tanh(acc2)

    p = sel_ref[0] @ p4[:, 0:D]
    for q in range(1, GQ):
        p = p + sel_ref[q] @ p4[:, q * D:(q + 1) * D]
    z = lax.dot_general(p, item_ref[...], (((1,), (1,)), ((), ())))
    z = z[:, :M_ITEMS]
    z = z - jnp.max(z, axis=-1, keepdims=True)
    ez = jnp.exp(z)
    out_ref[...] = ez / jnp.sum(ez, axis=-1, keepdims=True)


def _full_spec(shape):
    return pl.BlockSpec(shape, lambda i: tuple(0 for _ in shape))


_TC_IN_SPECS = [
        pl.BlockSpec((MAXLEN, GB, 128), lambda i: (0, i, 0)),
        pl.BlockSpec((GB, GQ * DENSE), lambda i: (i, 0)),
        pl.BlockSpec((N_SPARSE, GB, 128), lambda i: (0, i, 0)),
        _full_spec((ITEM_PAD, D)),
        _full_spec((GQ, BB, GB)),
        _full_spec((1, 128)),
        _full_spec((128, 128)),
        _full_spec((128, 128)),
        _full_spec((128, 128)),
        _full_spec((1, 128)),
        _full_spec((128, 128)),
        _full_spec((128, 128)),
        _full_spec((GQ * DENSE, 128)),
        _full_spec((N_SPARSE, 128, 128)),
        _full_spec((1, 128)),
        _full_spec((128, 128)),
        _full_spec((GQ * DENSE, 128)),
        _full_spec((N_SPARSE, 128, 128)),
        _full_spec((1, 128)),
]

_tc_dense = pl.pallas_call(
    _tc_dense_body,
    grid=(G,),
    in_specs=_TC_IN_SPECS,
    out_specs=pl.BlockSpec((BB, M_ITEMS), lambda i: (i, 0)),
    out_shape=jax.ShapeDtypeStruct((B, M_ITEMS), jnp.float32),
)


def _blockdiag4(w):
    a, bdim = w.shape
    out = jnp.zeros((GQ * a, GQ * bdim), jnp.float32)
    for q in range(GQ):
        out = lax.dynamic_update_slice(out, w, (q * a, q * bdim))
    return out


@jax.jit
def kernel(dense_inputs, sparse_inputs, seq_inputs, item_pooling, table_sparse,
           table_seq, W0, W1, W2, W3, b, ffn1_W, ffn1_b, ffn2_W, ffn2_b):
    seq_idx = seq_inputs[:, 0, :].T.reshape(-1)
    feat_off = (jnp.arange(N_SPARSE, dtype=jnp.int32) * SPARSE_VOCAB)[:, None]
    sparse_idx = (sparse_inputs.T + feat_off).reshape(-1)
    item_idx = jnp.concatenate(
        [item_pooling[:, 0], jnp.zeros((ITEM_PAD - M_ITEMS,), jnp.int32)])
    table_sparse_flat = table_sparse.reshape(N_SPARSE * SPARSE_VOCAB, D)

    seq_flat, sparse_flat, item_embed = _get_sc_gather()(
        seq_idx, sparse_idx, item_idx, table_seq, table_sparse_flat)

    seq_pk = seq_flat.reshape(MAXLEN, BL, 128)
    sparse_pk = sparse_flat.reshape(N_SPARSE, BL, 128)
    dense_pk = dense_inputs.reshape(BL, GQ * DENSE)

    w1_4 = _blockdiag4(W1)
    w2_4 = _blockdiag4(W2)
    w3_4 = _blockdiag4(W3)
    w0_4 = jnp.tile(W0.T, (1, GQ))
    b_4 = jnp.tile(b[None, :], (1, GQ))
    gones = _blockdiag4(jnp.ones((D, D), jnp.float32))
    f1m_4 = _blockdiag4(ffn1_W[:D])
    f1d_4 = _blockdiag4(ffn1_W[D:D + DENSE])
    f1b_4 = jnp.tile(ffn1_b[None, :], (1, GQ))
    f2m_4 = _blockdiag4(ffn2_W[:D])
    f2d_4 = _blockdiag4(ffn2_W[D:D + DENSE])
    f2b_4 = jnp.tile(ffn2_b[None, :], (1, GQ))
    f1s = ffn1_W[D + DENSE:].reshape(N_SPARSE, D, D)
    f2s = ffn2_W[D + DENSE:].reshape(N_SPARSE, D, D)
    f1s_4 = jnp.stack([_blockdiag4(f1s[i]) for i in range(N_SPARSE)])
    f2s_4 = jnp.stack([_blockdiag4(f2s[i]) for i in range(N_SPARSE)])
    rows = jnp.arange(BB)
    cols = jnp.arange(GB)
    sel = jnp.stack([
        (rows[:, None] == cols[None, :] * GQ + q).astype(jnp.float32)
        for q in range(GQ)])

    return _tc_dense(
        seq_pk, dense_pk, sparse_pk, item_embed, sel,
        w0_4, w1_4, w2_4, w3_4, b_4, gones,
        f1m_4, f1d_4, f1s_4, f1b_4,
        f2m_4, f2d_4, f2s_4, f2b_4)
